# baseline (speedup 1.0000x reference)
import functools

import jax
import jax.numpy as jnp
from jax import lax
from jax.experimental import pallas as pl
from jax.experimental.pallas import tpu as pltpu
from jax.experimental.pallas import tpu_sc as plsc

N_NODES = 10000
D = 128
NC = 2
NS = 16
L = 16
NT = NC * NS
B = 128
NB = 82
EPT = NB * B
E_CAP = NT * EPT
N_PAD = 10240
RS = N_PAD // NS
CROWS = N_PAD // L
CCH = CROWS // B

_mesh = plsc.VectorSubcoreMesh(core_axis_name="core", subcore_axis_name="subcore")
_sc_params = pltpu.CompilerParams(needs_layout_passes=False)


RC_SHIFT = 14
RC_MASK = (1 << RC_SHIFT) - 1
NSUB = 4


def _deg_body(pk_hbm, cnt_hbm, pk_v, cnt_v, idx_v, cnt_s):
    cid = lax.axis_index("core")
    sid = lax.axis_index("subcore")
    t = cid * NS + sid

    zeros16 = jnp.zeros((L,), jnp.float32)
    ones16 = jnp.ones((L,), jnp.float32)

    @pl.loop(0, CROWS)
    def _(r):
        cnt_v[r, :] = zeros16

    for c in range(CCH):
        @pl.loop(0, B, step=L)
        def _(k, c=c):
            idx_v[c, pl.ds(k, L)] = lax.iota(jnp.int32, L) + (c * B + k)

    @pl.when(sid == 0)
    def _():
        pltpu.sync_copy(cnt_v, cnt_s)

    pltpu.sync_copy(pk_hbm.at[t], pk_v)

    @pl.loop(0, EPT, step=L)
    def _(i):
        idx = pk_v[pl.ds(i, L)] & RC_MASK
        plsc.addupdate_scatter(cnt_v, [idx >> 4, idx & 15], ones16)

    plsc.subcore_barrier()
    for c in range(CCH):
        pltpu.sync_copy(cnt_v.at[pl.ds(c * B, B)], cnt_s.at[idx_v.at[c]],
                        add=True)
    plsc.subcore_barrier()
    pltpu.sync_copy(cnt_s.at[pl.ds(sid * (CROWS // NS), CROWS // NS)],
                    cnt_hbm.at[cid, pl.ds(sid * (CROWS // NS), CROWS // NS)])


@jax.jit
def _deg_call(pk_p):
    k = pl.kernel(
        _deg_body,
        out_type=jax.ShapeDtypeStruct((NC, CROWS, L), jnp.float32),
        mesh=_mesh,
        scratch_types=[
            pltpu.VMEM((EPT,), jnp.int32),
            pltpu.VMEM((CROWS, L), jnp.float32),
            pltpu.VMEM((CCH, B), jnp.int32),
            pltpu.VMEM_SHARED((CROWS, L), jnp.float32),
        ],
        compiler_params=_sc_params,
    )
    return k(pk_p)


def _msg_body(y_hbm, pk_hbm, zero_hbm, p_hbm,
              pk_v, row_b, col_b, buf0, buf1, acc_s, gsem0, gsem1):
    cid = lax.axis_index("core")
    sid = lax.axis_index("subcore")
    t = cid * NS + sid

    pltpu.sync_copy(zero_hbm.at[pl.ds(sid * RS, RS)],
                    acc_s.at[pl.ds(sid * RS, RS)])
    pltpu.sync_copy(pk_hbm.at[t], pk_v)
    plsc.subcore_barrier()

    def unpack(j, slot):
        for q in range(NSUB):
            @pl.loop(0, B // NSUB, step=L)
            def _(k, q=q):
                pk = pk_v[j, pl.ds(q * (B // NSUB) + k, L)]
                row_b[slot, q, pl.ds(k, L)] = pk >> RC_SHIFT
                col_b[slot, pl.ds(q * (B // NSUB) + k, L)] = pk & RC_MASK

    def start(buf, slot, sem):
        for q in range(NSUB):
            pltpu.async_copy(y_hbm.at[row_b.at[slot, q]],
                             buf.at[pl.ds(q * (B // NSUB), B // NSUB)], sem)

    def wait(buf, sem):
        pltpu.make_async_copy(y_hbm.at[pl.ds(0, B)], buf, sem).wait()

    def scat(buf, slot):
        pltpu.sync_copy(buf, acc_s.at[col_b.at[slot]], add=True)

    unpack(0, 0)
    start(buf0, 0, gsem0)

    @pl.loop(0, NB, step=2)
    def _(j):
        unpack(j + 1, 1)
        start(buf1, 1, gsem1)
        wait(buf0, gsem0)
        scat(buf0, 0)

        @pl.when(j + 2 < NB)
        def _():
            unpack(j + 2, 0)
            start(buf0, 0, gsem0)

        wait(buf1, gsem1)
        scat(buf1, 1)

    plsc.subcore_barrier()
    pltpu.sync_copy(acc_s.at[pl.ds(sid * RS, RS)],
                    p_hbm.at[cid, pl.ds(sid * RS, RS)])


@jax.jit
def _msg_call(y, pk_p, zeros):
    k = pl.kernel(
        _msg_body,
        out_type=jax.ShapeDtypeStruct((NC, N_PAD, D), jnp.float32),
        mesh=_mesh,
        scratch_types=[
            pltpu.VMEM((NB, B), jnp.int32),
            pltpu.VMEM((2, NSUB, B // NSUB), jnp.int32),
            pltpu.VMEM((2, B), jnp.int32),
            pltpu.VMEM((B, D), jnp.float32),
            pltpu.VMEM((B, D), jnp.float32),
            pltpu.VMEM_SHARED((N_PAD, D), jnp.float32),
            pltpu.SemaphoreType.DMA,
            pltpu.SemaphoreType.DMA,
        ],
        compiler_params=_sc_params,
    )
    return k(y, pk_p, zeros)


ROWS_BLK = 400
GRID = N_NODES // ROWS_BLK


def _mm_body(x_ref, w_ref, xt_ref):
    xt_ref[...] = lax.dot_general(
        x_ref[...], w_ref[...], (((1,), (1,)), ((), ())),
        preferred_element_type=jnp.float32,
        precision=lax.Precision.HIGHEST)


def _scale_body(xt_ref, cnt_ref, y_ref):
    deg = cnt_ref[0] + cnt_ref[1]
    dis = lax.rsqrt(deg)
    y_ref[...] = dis * xt_ref[...]


def _final_body(p_ref, cnt_ref, bias_ref, o_ref):
    deg = cnt_ref[0] + cnt_ref[1]
    dis = lax.rsqrt(deg)
    o_ref[...] = dis * (p_ref[0] + p_ref[1]) + bias_ref[...]


@jax.jit
def _tc_mm(x, W):
    return pl.pallas_call(
        _mm_body,
        grid=(GRID,),
        in_specs=[
            pl.BlockSpec((ROWS_BLK, D), lambda i: (i, 0)),
            pl.BlockSpec((D, D), lambda i: (0, 0)),
        ],
        out_specs=pl.BlockSpec((ROWS_BLK, D), lambda i: (i, 0)),
        out_shape=jax.ShapeDtypeStruct((N_NODES, D), jnp.float32),
    )(x, W)


@jax.jit
def _tc_scale(xt, cnt):
    return pl.pallas_call(
        _scale_body,
        grid=(GRID,),
        in_specs=[
            pl.BlockSpec((ROWS_BLK, D), lambda i: (i, 0)),
            pl.BlockSpec((NC, ROWS_BLK, 1), lambda i: (0, i, 0)),
        ],
        out_specs=pl.BlockSpec((ROWS_BLK, D), lambda i: (i, 0)),
        out_shape=jax.ShapeDtypeStruct((N_NODES, D), jnp.float32),
    )(xt, cnt)


@jax.jit
def _tc_final(p, cnt, bias):
    return pl.pallas_call(
        _final_body,
        grid=(GRID,),
        in_specs=[
            pl.BlockSpec((NC, ROWS_BLK, D), lambda i: (0, i, 0)),
            pl.BlockSpec((NC, ROWS_BLK, 1), lambda i: (0, i, 0)),
            pl.BlockSpec((1, D), lambda i: (0, 0)),
        ],
        out_specs=pl.BlockSpec((ROWS_BLK, D), lambda i: (i, 0)),
        out_shape=jax.ShapeDtypeStruct((N_NODES, D), jnp.float32),
    )(p, cnt, bias)


def kernel(x, edge_index, W, bias):
    N = x.shape[0]
    loops = jnp.arange(N, dtype=edge_index.dtype)
    row = jnp.concatenate([edge_index[0], loops]).astype(jnp.int32)
    col = jnp.concatenate([edge_index[1], loops]).astype(jnp.int32)
    e_tot = row.shape[0]
    pad = E_CAP - e_tot
    packed = (row << RC_SHIFT) | col
    pad_pos = jnp.arange(pad, dtype=jnp.int32) + e_tot
    pad_col = N_NODES + (pad_pos % NS) * 15 + (pad_pos // NT) % 15
    pk_p = (jnp.concatenate([packed, pad_col])
            .reshape(EPT, NT).T.reshape(NT, NB, B))

    cnt = _deg_call(pk_p.reshape(NT, EPT)).reshape(NC, N_PAD, 1)
    xt = _tc_mm(x, W)
    y = _tc_scale(xt, cnt)
    zeros = jnp.zeros((N_PAD, D), jnp.float32)
    p = _msg_call(y, pk_p, zeros)
    out = _tc_final(p, cnt, bias.reshape(1, D))
    return out


# speedup vs baseline: 1.3479x; 1.3479x over previous
import functools

import jax
import jax.numpy as jnp
from jax import lax
from jax.experimental import pallas as pl
from jax.experimental.pallas import tpu as pltpu
from jax.experimental.pallas import tpu_sc as plsc

N_NODES = 10000
D = 128
NC = 2
NS = 16
L = 16
NT = NC * NS
B = 128
NB = 82
EPT = NB * B
E_CAP = NT * EPT
N_PAD = 10240
RS = N_PAD // NS
CROWS = N_PAD // L
CCH = CROWS // B

_mesh = plsc.VectorSubcoreMesh(core_axis_name="core", subcore_axis_name="subcore")
_sc_params = pltpu.CompilerParams(needs_layout_passes=False,
                                  use_tc_tiling_on_sc=False)


RC_SHIFT = 14
RC_MASK = (1 << RC_SHIFT) - 1
NSUB = 4


def _deg_body(pk_hbm, cnt_hbm, pk_v, cnt_v, idx_v, cnt_s):
    cid = lax.axis_index("core")
    sid = lax.axis_index("subcore")
    t = cid * NS + sid

    zeros16 = jnp.zeros((L,), jnp.float32)
    ones16 = jnp.ones((L,), jnp.float32)

    @pl.loop(0, CROWS)
    def _(r):
        cnt_v[r, :] = zeros16

    for c in range(CCH):
        @pl.loop(0, B, step=L)
        def _(k, c=c):
            idx_v[c, pl.ds(k, L)] = lax.iota(jnp.int32, L) + (c * B + k)

    @pl.when(sid == 0)
    def _():
        pltpu.sync_copy(cnt_v, cnt_s)

    pltpu.sync_copy(pk_hbm.at[t], pk_v)

    @pl.loop(0, EPT, step=L)
    def _(i):
        idx = pk_v[pl.ds(i, L)] & RC_MASK
        plsc.addupdate_scatter(cnt_v, [idx >> 4, idx & 15], ones16)

    plsc.subcore_barrier()
    for c in range(CCH):
        pltpu.sync_copy(cnt_v.at[pl.ds(c * B, B)], cnt_s.at[idx_v.at[c]],
                        add=True)
    plsc.subcore_barrier()
    pltpu.sync_copy(cnt_s.at[pl.ds(sid * (CROWS // NS), CROWS // NS)],
                    cnt_hbm.at[cid, pl.ds(sid * (CROWS // NS), CROWS // NS)])


@jax.jit
def _deg_call(pk_p):
    k = pl.kernel(
        _deg_body,
        out_type=jax.ShapeDtypeStruct((NC, CROWS, L), jnp.float32),
        mesh=_mesh,
        scratch_types=[
            pltpu.VMEM((EPT,), jnp.int32),
            pltpu.VMEM((CROWS, L), jnp.float32),
            pltpu.VMEM((CCH, B), jnp.int32),
            pltpu.VMEM_SHARED((CROWS, L), jnp.float32),
        ],
        compiler_params=_sc_params,
    )
    return k(pk_p)


def _msg_body(y_hbm, pk_hbm, zero_hbm, p_hbm,
              pk_v, row_b, col_b, buf0, buf1, acc_s, gsem0, gsem1):
    cid = lax.axis_index("core")
    sid = lax.axis_index("subcore")
    t = cid * NS + sid

    pltpu.sync_copy(zero_hbm.at[pl.ds(sid * RS, RS)],
                    acc_s.at[pl.ds(sid * RS, RS)])
    pltpu.sync_copy(pk_hbm.at[t], pk_v)
    plsc.subcore_barrier()

    def unpack(j, slot):
        for q in range(NSUB):
            @pl.loop(0, B // NSUB, step=L)
            def _(k, q=q):
                pk = pk_v[j, pl.ds(q * (B // NSUB) + k, L)]
                row_b[slot, q, pl.ds(k, L)] = pk >> RC_SHIFT
                col_b[slot, pl.ds(q * (B // NSUB) + k, L)] = pk & RC_MASK

    def start(buf, slot, sem):
        for q in range(NSUB):
            pltpu.async_copy(y_hbm.at[row_b.at[slot, q]],
                             buf.at[pl.ds(q * (B // NSUB), B // NSUB)], sem)

    def wait(buf, sem):
        pltpu.make_async_copy(y_hbm.at[pl.ds(0, B)], buf, sem).wait()

    def scat(buf, slot):
        pltpu.sync_copy(buf, acc_s.at[col_b.at[slot]], add=True)

    unpack(0, 0)
    start(buf0, 0, gsem0)

    @pl.loop(0, NB, step=2)
    def _(j):
        unpack(j + 1, 1)
        start(buf1, 1, gsem1)
        wait(buf0, gsem0)
        scat(buf0, 0)

        @pl.when(j + 2 < NB)
        def _():
            unpack(j + 2, 0)
            start(buf0, 0, gsem0)

        wait(buf1, gsem1)
        scat(buf1, 1)

    plsc.subcore_barrier()
    pltpu.sync_copy(acc_s.at[pl.ds(sid * RS, RS)],
                    p_hbm.at[cid, pl.ds(sid * RS, RS)])


@jax.jit
def _msg_call(y, pk_p, zeros):
    k = pl.kernel(
        _msg_body,
        out_type=jax.ShapeDtypeStruct((NC, N_PAD, D), jnp.bfloat16),
        mesh=_mesh,
        scratch_types=[
            pltpu.VMEM((NB, B), jnp.int32),
            pltpu.VMEM((2, NSUB, B // NSUB), jnp.int32),
            pltpu.VMEM((2, B), jnp.int32),
            pltpu.VMEM((B, D), jnp.bfloat16),
            pltpu.VMEM((B, D), jnp.bfloat16),
            pltpu.VMEM_SHARED((N_PAD, D), jnp.bfloat16),
            pltpu.SemaphoreType.DMA,
            pltpu.SemaphoreType.DMA,
        ],
        compiler_params=_sc_params,
    )
    return k(y, pk_p, zeros)


ROWS_BLK = 400
GRID = N_NODES // ROWS_BLK


def _mm_body(x_ref, w_ref, xt_ref):
    xt_ref[...] = lax.dot_general(
        x_ref[...], w_ref[...], (((1,), (1,)), ((), ())),
        preferred_element_type=jnp.float32,
        precision=lax.Precision.HIGHEST)


def _scale_body(xt_ref, cnt_ref, y_ref):
    deg = cnt_ref[0] + cnt_ref[1]
    dis = lax.rsqrt(deg)
    y_ref[...] = (dis * xt_ref[...]).astype(jnp.bfloat16)


def _final_body(p_ref, cnt_ref, bias_ref, o_ref):
    deg = cnt_ref[0] + cnt_ref[1]
    dis = lax.rsqrt(deg)
    s = p_ref[0].astype(jnp.float32) + p_ref[1].astype(jnp.float32)
    o_ref[...] = dis * s + bias_ref[...]


@jax.jit
def _tc_mm(x, W):
    return pl.pallas_call(
        _mm_body,
        grid=(GRID,),
        in_specs=[
            pl.BlockSpec((ROWS_BLK, D), lambda i: (i, 0)),
            pl.BlockSpec((D, D), lambda i: (0, 0)),
        ],
        out_specs=pl.BlockSpec((ROWS_BLK, D), lambda i: (i, 0)),
        out_shape=jax.ShapeDtypeStruct((N_NODES, D), jnp.float32),
    )(x, W)


@jax.jit
def _tc_scale(xt, cnt):
    return pl.pallas_call(
        _scale_body,
        grid=(GRID,),
        in_specs=[
            pl.BlockSpec((ROWS_BLK, D), lambda i: (i, 0)),
            pl.BlockSpec((NC, ROWS_BLK, 1), lambda i: (0, i, 0)),
        ],
        out_specs=pl.BlockSpec((ROWS_BLK, D), lambda i: (i, 0)),
        out_shape=jax.ShapeDtypeStruct((N_NODES, D), jnp.bfloat16),
    )(xt, cnt)


@jax.jit
def _tc_final(p, cnt, bias):
    return pl.pallas_call(
        _final_body,
        grid=(GRID,),
        in_specs=[
            pl.BlockSpec((NC, ROWS_BLK, D), lambda i: (0, i, 0)),
            pl.BlockSpec((NC, ROWS_BLK, 1), lambda i: (0, i, 0)),
            pl.BlockSpec((1, D), lambda i: (0, 0)),
        ],
        out_specs=pl.BlockSpec((ROWS_BLK, D), lambda i: (i, 0)),
        out_shape=jax.ShapeDtypeStruct((N_NODES, D), jnp.float32),
    )(p, cnt, bias)


def kernel(x, edge_index, W, bias):
    N = x.shape[0]
    loops = jnp.arange(N, dtype=edge_index.dtype)
    row = jnp.concatenate([edge_index[0], loops]).astype(jnp.int32)
    col = jnp.concatenate([edge_index[1], loops]).astype(jnp.int32)
    e_tot = row.shape[0]
    pad = E_CAP - e_tot
    packed = (row << RC_SHIFT) | col
    pad_pos = jnp.arange(pad, dtype=jnp.int32) + e_tot
    pad_col = N_NODES + (pad_pos % NS) * 15 + (pad_pos // NT) % 15
    pk_p = (jnp.concatenate([packed, pad_col])
            .reshape(EPT, NT).T.reshape(NT, NB, B))

    cnt = _deg_call(pk_p.reshape(NT, EPT)).reshape(NC, N_PAD, 1)
    xt = _tc_mm(x, W)
    y = _tc_scale(xt, cnt)
    zeros = jnp.zeros((N_PAD, D), jnp.bfloat16)
    p = _msg_call(y, pk_p, zeros)
    out = _tc_final(p, cnt, bias.reshape(1, D))
    return out


# speedup vs baseline: 2.3605x; 1.7512x over previous
import jax
import jax.numpy as jnp
from jax import lax
from jax.experimental import pallas as pl
from jax.experimental.pallas import tpu as pltpu
from jax.experimental.pallas import tpu_sc as plsc

N_NODES = 10000
D = 128
NC = 2
NS = 16
L = 16
NT = NC * NS
B = 125
NB = 80
EPT = NB * B
RS = N_NODES // NS
CROWS = 640
CCH = CROWS // 128

_mesh = plsc.VectorSubcoreMesh(core_axis_name="core", subcore_axis_name="subcore")
_sc_params = pltpu.CompilerParams(needs_layout_passes=False,
                                  use_tc_tiling_on_sc=False)


def _deg_body(col_hbm, cnt_hbm, col_v, cnt_v, idx_v, cnt_s):
    cid = lax.axis_index("core")
    sid = lax.axis_index("subcore")
    t = cid * NS + sid

    zeros16 = jnp.zeros((L,), jnp.float32)
    ones16 = jnp.ones((L,), jnp.float32)

    @pl.loop(0, CROWS)
    def _(r):
        cnt_v[r, :] = zeros16

    for c in range(CCH):
        @pl.loop(0, 128, step=L)
        def _(k, c=c):
            idx_v[c, pl.ds(k, L)] = lax.iota(jnp.int32, L) + (c * 128 + k)

    @pl.when(sid == 0)
    def _():
        pltpu.sync_copy(cnt_v, cnt_s)

    pltpu.sync_copy(col_hbm.at[t], col_v)

    @pl.loop(0, EPT, step=L)
    def _(i):
        idx = col_v[pl.ds(i, L)]
        plsc.addupdate_scatter(cnt_v, [idx >> 4, idx & 15], ones16)

    plsc.subcore_barrier()
    for c in range(CCH):
        pltpu.sync_copy(cnt_v.at[pl.ds(c * 128, 128)], cnt_s.at[idx_v.at[c]],
                        add=True)
    plsc.subcore_barrier()
    pltpu.sync_copy(cnt_s.at[pl.ds(sid * (CROWS // NS), CROWS // NS)],
                    cnt_hbm.at[cid, pl.ds(sid * (CROWS // NS), CROWS // NS)])


@jax.jit
def _deg_call(col_p):
    k = pl.kernel(
        _deg_body,
        out_type=jax.ShapeDtypeStruct((NC, CROWS, L), jnp.float32),
        mesh=_mesh,
        scratch_types=[
            pltpu.VMEM((EPT,), jnp.int32),
            pltpu.VMEM((CROWS, L), jnp.float32),
            pltpu.VMEM((CCH, 128), jnp.int32),
            pltpu.VMEM_SHARED((CROWS, L), jnp.float32),
        ],
        compiler_params=_sc_params,
    )
    return k(col_p)


def _msg_body(y_hbm, row_hbm, col_hbm, zero_hbm, p_hbm,
              row_v, col_v, buf0, buf1, acc_s, gsem0, gsem1):
    cid = lax.axis_index("core")
    sid = lax.axis_index("subcore")
    t = cid * NS + sid

    @pl.when(cid == 0)
    def _():
        pltpu.sync_copy(y_hbm.at[pl.ds(sid * RS, RS)],
                        acc_s.at[pl.ds(sid * RS, RS)])

    @pl.when(cid != 0)
    def _():
        pltpu.sync_copy(zero_hbm.at[pl.ds(sid * RS, RS)],
                        acc_s.at[pl.ds(sid * RS, RS)])

    pltpu.sync_copy(row_hbm.at[t], row_v)
    pltpu.sync_copy(col_hbm.at[t], col_v)
    plsc.subcore_barrier()

    def start(j, buf, sem):
        pltpu.async_copy(y_hbm.at[row_v.at[j]], buf, sem)

    def wait(buf, sem):
        pltpu.make_async_copy(y_hbm.at[pl.ds(0, B)], buf, sem).wait()

    def scat(j, buf):
        pltpu.sync_copy(buf, acc_s.at[col_v.at[j]], add=True)

    start(0, buf0, gsem0)

    @pl.loop(0, NB, step=2)
    def _(j):
        start(j + 1, buf1, gsem1)
        wait(buf0, gsem0)
        scat(j, buf0)

        @pl.when(j + 2 < NB)
        def _():
            start(j + 2, buf0, gsem0)

        wait(buf1, gsem1)
        scat(j + 1, buf1)

    plsc.subcore_barrier()
    pltpu.sync_copy(acc_s.at[pl.ds(sid * RS, RS)],
                    p_hbm.at[cid, pl.ds(sid * RS, RS)])


@jax.jit
def _msg_call(y, row_p, col_p, zeros):
    k = pl.kernel(
        _msg_body,
        out_type=jax.ShapeDtypeStruct((NC, N_NODES, D), jnp.bfloat16),
        mesh=_mesh,
        scratch_types=[
            pltpu.VMEM((NB, B), jnp.int32),
            pltpu.VMEM((NB, B), jnp.int32),
            pltpu.VMEM((B, D), jnp.bfloat16),
            pltpu.VMEM((B, D), jnp.bfloat16),
            pltpu.VMEM_SHARED((N_NODES, D), jnp.bfloat16),
            pltpu.SemaphoreType.DMA,
            pltpu.SemaphoreType.DMA,
        ],
        compiler_params=_sc_params,
    )
    return k(y, row_p, col_p, zeros)


ROWS_BLK = 2000
GRID = N_NODES // ROWS_BLK


def _mm_body(x_ref, w_ref, xt_ref):
    xt_ref[...] = lax.dot_general(
        x_ref[...], w_ref[...], (((1,), (1,)), ((), ())),
        preferred_element_type=jnp.float32,
        precision=lax.Precision.HIGHEST)


def _scale_body(xt_ref, cnt_ref, y_ref):
    deg = cnt_ref[0] + cnt_ref[1] + 1.0
    dis = lax.rsqrt(deg)
    y_ref[...] = (dis * xt_ref[...]).astype(jnp.bfloat16)


def _final_body(p_ref, cnt_ref, bias_ref, o_ref):
    deg = cnt_ref[0] + cnt_ref[1] + 1.0
    dis = lax.rsqrt(deg)
    s = p_ref[0].astype(jnp.float32) + p_ref[1].astype(jnp.float32)
    o_ref[...] = dis * s + bias_ref[...]


@jax.jit
def _tc_mm(x, W):
    return pl.pallas_call(
        _mm_body,
        grid=(GRID,),
        in_specs=[
            pl.BlockSpec((ROWS_BLK, D), lambda i: (i, 0)),
            pl.BlockSpec((D, D), lambda i: (0, 0)),
        ],
        out_specs=pl.BlockSpec((ROWS_BLK, D), lambda i: (i, 0)),
        out_shape=jax.ShapeDtypeStruct((N_NODES, D), jnp.float32),
    )(x, W)


@jax.jit
def _tc_scale(xt, cnt):
    return pl.pallas_call(
        _scale_body,
        grid=(GRID,),
        in_specs=[
            pl.BlockSpec((ROWS_BLK, D), lambda i: (i, 0)),
            pl.BlockSpec((NC, ROWS_BLK, 1), lambda i: (0, i, 0)),
        ],
        out_specs=pl.BlockSpec((ROWS_BLK, D), lambda i: (i, 0)),
        out_shape=jax.ShapeDtypeStruct((N_NODES, D), jnp.bfloat16),
    )(xt, cnt)


@jax.jit
def _tc_final(p, cnt, bias):
    return pl.pallas_call(
        _final_body,
        grid=(GRID,),
        in_specs=[
            pl.BlockSpec((NC, ROWS_BLK, D), lambda i: (0, i, 0)),
            pl.BlockSpec((NC, ROWS_BLK, 1), lambda i: (0, i, 0)),
            pl.BlockSpec((1, D), lambda i: (0, 0)),
        ],
        out_specs=pl.BlockSpec((ROWS_BLK, D), lambda i: (i, 0)),
        out_shape=jax.ShapeDtypeStruct((N_NODES, D), jnp.float32),
    )(p, cnt, bias)


def kernel(x, edge_index, W, bias):
    row = edge_index[0].astype(jnp.int32)
    col = edge_index[1].astype(jnp.int32)
    row_p = row.reshape(NT, NB, B)
    col_p = col.reshape(NT, NB, B)

    cnt = _deg_call(col.reshape(NT, EPT)).reshape(NC, NS * CROWS, 1)
    xt = _tc_mm(x, W)
    y = _tc_scale(xt, cnt)
    zeros = jnp.zeros((N_NODES, D), jnp.bfloat16)
    p = _msg_call(y, row_p, col_p, zeros)
    out = _tc_final(p, cnt, bias.reshape(1, D))
    return out


# speedup vs baseline: 2.5491x; 1.0799x over previous
import jax
import jax.numpy as jnp
from jax import lax
from jax.experimental import pallas as pl
from jax.experimental.pallas import tpu as pltpu
from jax.experimental.pallas import tpu_sc as plsc

N_NODES = 10000
D = 128
NC = 2
NS = 16
L = 16
NT = NC * NS
B = 125
NB = 80
EPT = NB * B
RS = N_NODES // NS
CROWS = 640
CCH = CROWS // 128

_mesh = plsc.VectorSubcoreMesh(core_axis_name="core", subcore_axis_name="subcore")
_sc_params = pltpu.CompilerParams(needs_layout_passes=False,
                                  use_tc_tiling_on_sc=False)


def _deg_body(col_hbm, cnt_hbm, col_v, cnt_v, idx_v, red_v, cnt_s):
    cid = lax.axis_index("core")
    sid = lax.axis_index("subcore")
    t = cid * NS + sid

    zeros16 = jnp.zeros((L,), jnp.float32)
    ones16 = jnp.ones((L,), jnp.float32)

    @pl.loop(0, CROWS)
    def _(r):
        cnt_v[r, :] = zeros16

    for c in range(CCH):
        @pl.loop(0, 128, step=L)
        def _(k, c=c):
            idx_v[c, pl.ds(k, L)] = lax.iota(jnp.int32, L) + (c * 128 + k)

    @pl.when(sid == 0)
    def _():
        pltpu.sync_copy(cnt_v, cnt_s)

    pltpu.sync_copy(col_hbm.at[t], col_v)

    @pl.loop(0, EPT, step=L)
    def _(i):
        idx = col_v[pl.ds(i, L)]
        plsc.addupdate_scatter(cnt_v, [idx >> 4, idx & 15], ones16)

    plsc.subcore_barrier()
    for c in range(CCH):
        pltpu.sync_copy(cnt_v.at[pl.ds(c * 128, 128)], cnt_s.at[idx_v.at[c]],
                        add=True)
    plsc.subcore_barrier()
    nsh = CROWS // NS
    pltpu.sync_copy(cnt_s.at[pl.ds(sid * nsh, nsh)], cnt_v.at[pl.ds(0, nsh)])

    @pl.loop(0, nsh)
    def _(r):
        red_v[pl.ds(r * L, L)] = cnt_v[r, :]

    pltpu.sync_copy(red_v, cnt_hbm.at[cid, pl.ds(sid * (nsh * L), nsh * L)])


@jax.jit
def _deg_call(col_p):
    k = pl.kernel(
        _deg_body,
        out_type=jax.ShapeDtypeStruct((NC, CROWS * L), jnp.float32),
        mesh=_mesh,
        scratch_types=[
            pltpu.VMEM((EPT,), jnp.int32),
            pltpu.VMEM((CROWS, L), jnp.float32),
            pltpu.VMEM((CCH, 128), jnp.int32),
            pltpu.VMEM((CROWS // NS * L,), jnp.float32),
            pltpu.VMEM_SHARED((CROWS, L), jnp.float32),
        ],
        compiler_params=_sc_params,
    )
    return k(col_p)


def _msg_body(y_hbm, row_hbm, col_hbm, zero_hbm, p_hbm,
              row_v, col_v, buf0, buf1, acc_s, gsem0, gsem1):
    cid = lax.axis_index("core")
    sid = lax.axis_index("subcore")
    t = cid * NS + sid

    @pl.when(cid == 0)
    def _():
        pltpu.sync_copy(y_hbm.at[pl.ds(sid * RS, RS)],
                        acc_s.at[pl.ds(sid * RS, RS)])

    @pl.when(cid != 0)
    def _():
        pltpu.sync_copy(zero_hbm.at[pl.ds(sid * RS, RS)],
                        acc_s.at[pl.ds(sid * RS, RS)])

    pltpu.sync_copy(row_hbm.at[t], row_v)
    pltpu.sync_copy(col_hbm.at[t], col_v)
    plsc.subcore_barrier()

    def start(j, buf, sem):
        pltpu.async_copy(y_hbm.at[row_v.at[j]], buf, sem)

    def wait(buf, sem):
        pltpu.make_async_copy(y_hbm.at[pl.ds(0, B)], buf, sem).wait()

    def scat(j, buf):
        pltpu.sync_copy(buf, acc_s.at[col_v.at[j]], add=True)

    start(0, buf0, gsem0)

    @pl.loop(0, NB, step=2)
    def _(j):
        start(j + 1, buf1, gsem1)
        wait(buf0, gsem0)
        scat(j, buf0)

        @pl.when(j + 2 < NB)
        def _():
            start(j + 2, buf0, gsem0)

        wait(buf1, gsem1)
        scat(j + 1, buf1)

    plsc.subcore_barrier()
    pltpu.sync_copy(acc_s.at[pl.ds(sid * RS, RS)],
                    p_hbm.at[cid, pl.ds(sid * RS, RS)])


@jax.jit
def _msg_call(y, row_p, col_p, zeros):
    k = pl.kernel(
        _msg_body,
        out_type=jax.ShapeDtypeStruct((NC, N_NODES, D), jnp.bfloat16),
        mesh=_mesh,
        scratch_types=[
            pltpu.VMEM((NB, B), jnp.int32),
            pltpu.VMEM((NB, B), jnp.int32),
            pltpu.VMEM((B, D), jnp.bfloat16),
            pltpu.VMEM((B, D), jnp.bfloat16),
            pltpu.VMEM_SHARED((N_NODES, D), jnp.bfloat16),
            pltpu.SemaphoreType.DMA,
            pltpu.SemaphoreType.DMA,
        ],
        compiler_params=_sc_params,
    )
    return k(y, row_p, col_p, zeros)


ROWS_BLK = 2048
GRID = -(-N_NODES // ROWS_BLK)


def _mm_body(x_ref, w_ref, xt_ref):
    xt_ref[...] = lax.dot_general(
        x_ref[...], w_ref[...], (((1,), (1,)), ((), ())),
        preferred_element_type=jnp.float32,
        precision=lax.Precision.HIGHEST)


def _dis_block(cnt_ref):
    s = pl.program_id(0) * ROWS_BLK
    deg = cnt_ref[0, pl.ds(s, ROWS_BLK)] + cnt_ref[1, pl.ds(s, ROWS_BLK)] + 1.0
    return jnp.reshape(lax.rsqrt(deg), (ROWS_BLK, 1))


def _scale_body(xt_ref, cnt_ref, y_ref):
    y_ref[...] = (_dis_block(cnt_ref) * xt_ref[...]).astype(jnp.bfloat16)


def _final_body(p_ref, cnt_ref, bias_ref, o_ref):
    s = p_ref[0].astype(jnp.float32) + p_ref[1].astype(jnp.float32)
    o_ref[...] = _dis_block(cnt_ref) * s + bias_ref[...]


@jax.jit
def _tc_mm(x, W):
    return pl.pallas_call(
        _mm_body,
        grid=(GRID,),
        in_specs=[
            pl.BlockSpec((ROWS_BLK, D), lambda i: (i, 0)),
            pl.BlockSpec((D, D), lambda i: (0, 0)),
        ],
        out_specs=pl.BlockSpec((ROWS_BLK, D), lambda i: (i, 0)),
        out_shape=jax.ShapeDtypeStruct((N_NODES, D), jnp.float32),
    )(x, W)


@jax.jit
def _tc_scale(xt, cnt):
    return pl.pallas_call(
        _scale_body,
        grid=(GRID,),
        in_specs=[
            pl.BlockSpec((ROWS_BLK, D), lambda i: (i, 0)),
            pl.BlockSpec((NC, CROWS * L), lambda i: (0, 0)),
        ],
        out_specs=pl.BlockSpec((ROWS_BLK, D), lambda i: (i, 0)),
        out_shape=jax.ShapeDtypeStruct((N_NODES, D), jnp.bfloat16),
    )(xt, cnt)


@jax.jit
def _tc_final(p, cnt, bias):
    return pl.pallas_call(
        _final_body,
        grid=(GRID,),
        in_specs=[
            pl.BlockSpec((NC, ROWS_BLK, D), lambda i: (0, i, 0)),
            pl.BlockSpec((NC, CROWS * L), lambda i: (0, 0)),
            pl.BlockSpec((1, D), lambda i: (0, 0)),
        ],
        out_specs=pl.BlockSpec((ROWS_BLK, D), lambda i: (i, 0)),
        out_shape=jax.ShapeDtypeStruct((N_NODES, D), jnp.float32),
    )(p, cnt, bias)


def kernel(x, edge_index, W, bias):
    row = edge_index[0].astype(jnp.int32)
    col = edge_index[1].astype(jnp.int32)
    row_p = row.reshape(NT, NB, B)
    col_p = col.reshape(NT, NB, B)

    cnt = _deg_call(col.reshape(NT, EPT))
    xt = _tc_mm(x, W)
    y = _tc_scale(xt, cnt)
    zeros = jnp.zeros((N_NODES, D), jnp.bfloat16)
    p = _msg_call(y, row_p, col_p, zeros)
    out = _tc_final(p, cnt, bias.reshape(1, D))
    return out


# speedup vs baseline: 2.6325x; 1.0327x over previous
import jax
import jax.numpy as jnp
from jax import lax
from jax.experimental import pallas as pl
from jax.experimental.pallas import tpu as pltpu
from jax.experimental.pallas import tpu_sc as plsc

N_NODES = 10000
D = 128
NC = 2
NS = 16
L = 16
NT = NC * NS
B = 80
NB = 125
EPT = NB * B
RS = N_NODES // NS
CROWS = 640
CCH = CROWS // 128

_mesh = plsc.VectorSubcoreMesh(core_axis_name="core", subcore_axis_name="subcore")
_sc_params = pltpu.CompilerParams(needs_layout_passes=False,
                                  use_tc_tiling_on_sc=False)


def _deg_body(ei_hbm, cnt_hbm, col_v, cnt_v, idx_v, red_v, cnt_s):
    cid = lax.axis_index("core")
    sid = lax.axis_index("subcore")
    t = cid * NS + sid

    zeros16 = jnp.zeros((L,), jnp.float32)
    ones16 = jnp.ones((L,), jnp.float32)

    @pl.loop(0, CROWS)
    def _(r):
        cnt_v[r, :] = zeros16

    for c in range(CCH):
        @pl.loop(0, 128, step=L)
        def _(k, c=c):
            idx_v[c, pl.ds(k, L)] = lax.iota(jnp.int32, L) + (c * 128 + k)

    @pl.when(sid == 0)
    def _():
        pltpu.sync_copy(cnt_v, cnt_s)

    pltpu.sync_copy(ei_hbm.at[1, pl.ds(t * EPT, EPT)], col_v)

    @pl.loop(0, EPT, step=L)
    def _(i):
        idx = col_v[pl.ds(i, L)]
        plsc.addupdate_scatter(cnt_v, [idx >> 4, idx & 15], ones16)

    plsc.subcore_barrier()
    for c in range(CCH):
        pltpu.sync_copy(cnt_v.at[pl.ds(c * 128, 128)], cnt_s.at[idx_v.at[c]],
                        add=True)
    plsc.subcore_barrier()
    nsh = CROWS // NS
    pltpu.sync_copy(cnt_s.at[pl.ds(sid * nsh, nsh)], cnt_v.at[pl.ds(0, nsh)])

    @pl.loop(0, nsh)
    def _(r):
        red_v[pl.ds(r * L, L)] = cnt_v[r, :]

    pltpu.sync_copy(red_v, cnt_hbm.at[cid, pl.ds(sid * (nsh * L), nsh * L)])


@jax.jit
def _deg_call(ei):
    k = pl.kernel(
        _deg_body,
        out_type=jax.ShapeDtypeStruct((NC, CROWS * L), jnp.float32),
        mesh=_mesh,
        scratch_types=[
            pltpu.VMEM((EPT,), jnp.int32),
            pltpu.VMEM((CROWS, L), jnp.float32),
            pltpu.VMEM((CCH, 128), jnp.int32),
            pltpu.VMEM((CROWS // NS * L,), jnp.float32),
            pltpu.VMEM_SHARED((CROWS, L), jnp.float32),
        ],
        compiler_params=_sc_params,
    )
    return k(ei)


def _msg_body(y_hbm, ei_hbm, zero_hbm, p_hbm,
              row_v, col_v, buf0, buf1, acc_s, gsem0, gsem1):
    cid = lax.axis_index("core")
    sid = lax.axis_index("subcore")
    t = cid * NS + sid

    @pl.when(cid == 0)
    def _():
        pltpu.sync_copy(y_hbm.at[pl.ds(sid * RS, RS)],
                        acc_s.at[pl.ds(sid * RS, RS)])

    @pl.when(cid != 0)
    def _():
        pltpu.sync_copy(zero_hbm.at[pl.ds(sid * RS, RS)],
                        acc_s.at[pl.ds(sid * RS, RS)])

    pltpu.sync_copy(ei_hbm.at[0, pl.ds(t * EPT, EPT)], row_v)
    pltpu.sync_copy(ei_hbm.at[1, pl.ds(t * EPT, EPT)], col_v)
    plsc.subcore_barrier()

    def start(j, buf, sem):
        pltpu.async_copy(y_hbm.at[row_v.at[pl.ds(j * B, B)]], buf, sem)

    def wait(buf, sem):
        pltpu.make_async_copy(y_hbm.at[pl.ds(0, B)], buf, sem).wait()

    def scat(j, buf):
        pltpu.sync_copy(buf, acc_s.at[col_v.at[pl.ds(j * B, B)]], add=True)

    start(0, buf0, gsem0)

    @pl.loop(0, NB - 1, step=2)
    def _(j):
        start(j + 1, buf1, gsem1)
        wait(buf0, gsem0)
        scat(j, buf0)

        @pl.when(j + 2 < NB)
        def _():
            start(j + 2, buf0, gsem0)

        wait(buf1, gsem1)
        scat(j + 1, buf1)

    wait(buf0, gsem0)
    scat(NB - 1, buf0)

    plsc.subcore_barrier()
    pltpu.sync_copy(acc_s.at[pl.ds(sid * RS, RS)],
                    p_hbm.at[cid, pl.ds(sid * RS, RS)])


@jax.jit
def _msg_call(y, ei, zeros):
    k = pl.kernel(
        _msg_body,
        out_type=jax.ShapeDtypeStruct((NC, N_NODES, D), jnp.bfloat16),
        mesh=_mesh,
        scratch_types=[
            pltpu.VMEM((EPT,), jnp.int32),
            pltpu.VMEM((EPT,), jnp.int32),
            pltpu.VMEM((B, D), jnp.bfloat16),
            pltpu.VMEM((B, D), jnp.bfloat16),
            pltpu.VMEM_SHARED((N_NODES, D), jnp.bfloat16),
            pltpu.SemaphoreType.DMA,
            pltpu.SemaphoreType.DMA,
        ],
        compiler_params=_sc_params,
    )
    return k(y, ei, zeros)


ROWS_BLK = 2048
GRID = -(-N_NODES // ROWS_BLK)


def _mm_body(x_ref, w_ref, xt_ref):
    xt_ref[...] = lax.dot_general(
        x_ref[...], w_ref[...], (((1,), (1,)), ((), ())),
        preferred_element_type=jnp.float32,
        precision=lax.Precision.HIGHEST)


def _dis_block(cnt_ref):
    s = pl.program_id(0) * ROWS_BLK
    deg = cnt_ref[0, pl.ds(s, ROWS_BLK)] + cnt_ref[1, pl.ds(s, ROWS_BLK)] + 1.0
    return jnp.reshape(lax.rsqrt(deg), (ROWS_BLK, 1))


def _scale_body(xt_ref, cnt_ref, y_ref):
    y_ref[...] = (_dis_block(cnt_ref) * xt_ref[...]).astype(jnp.bfloat16)


def _final_body(p_ref, cnt_ref, bias_ref, o_ref):
    s = p_ref[0].astype(jnp.float32) + p_ref[1].astype(jnp.float32)
    o_ref[...] = _dis_block(cnt_ref) * s + bias_ref[...]


@jax.jit
def _tc_mm(x, W):
    return pl.pallas_call(
        _mm_body,
        grid=(GRID,),
        in_specs=[
            pl.BlockSpec((ROWS_BLK, D), lambda i: (i, 0)),
            pl.BlockSpec((D, D), lambda i: (0, 0)),
        ],
        out_specs=pl.BlockSpec((ROWS_BLK, D), lambda i: (i, 0)),
        out_shape=jax.ShapeDtypeStruct((N_NODES, D), jnp.float32),
    )(x, W)


@jax.jit
def _tc_scale(xt, cnt):
    return pl.pallas_call(
        _scale_body,
        grid=(GRID,),
        in_specs=[
            pl.BlockSpec((ROWS_BLK, D), lambda i: (i, 0)),
            pl.BlockSpec((NC, CROWS * L), lambda i: (0, 0)),
        ],
        out_specs=pl.BlockSpec((ROWS_BLK, D), lambda i: (i, 0)),
        out_shape=jax.ShapeDtypeStruct((N_NODES, D), jnp.bfloat16),
    )(xt, cnt)


@jax.jit
def _tc_final(p, cnt, bias):
    return pl.pallas_call(
        _final_body,
        grid=(GRID,),
        in_specs=[
            pl.BlockSpec((NC, ROWS_BLK, D), lambda i: (0, i, 0)),
            pl.BlockSpec((NC, CROWS * L), lambda i: (0, 0)),
            pl.BlockSpec((1, D), lambda i: (0, 0)),
        ],
        out_specs=pl.BlockSpec((ROWS_BLK, D), lambda i: (i, 0)),
        out_shape=jax.ShapeDtypeStruct((N_NODES, D), jnp.float32),
    )(p, cnt, bias)


def kernel(x, edge_index, W, bias):
    ei = edge_index.astype(jnp.int32)
    cnt = _deg_call(ei)
    xt = _tc_mm(x, W)
    y = _tc_scale(xt, cnt)
    zeros = jnp.zeros((N_NODES, D), jnp.bfloat16)
    p = _msg_call(y, ei, zeros)
    out = _tc_final(p, cnt, bias.reshape(1, D))
    return out
